# baseline (speedup 1.0000x reference)
import functools

import jax
import jax.numpy as jnp
from jax import lax
from jax.experimental import pallas as pl
from jax.experimental.pallas import tpu as pltpu
from jax.experimental.pallas import tpu_sc as plsc

N_NODE = 1000000
EMBED_DIM = 16
BATCH = 16384

NUM_CORES = 2
NUM_SUBCORES = 16
NUM_LANES = 16
NW = NUM_CORES * NUM_SUBCORES
B_PER_W = BATCH // NW
NBLK = B_PER_W // NUM_LANES


def _sc_body(table, bias_tab, nid, nnid,
             score_out, embu_out, embv_out, bias_out,
             idx_u, idx_v, rows_u, rows_v, prod_vm,
             bias_vm, score_vm, sem_u, sem_v, sem_b):
    wid = lax.axis_index("s") * NUM_CORES + lax.axis_index("c")
    base = wid * B_PER_W

    pltpu.sync_copy(nid.at[pl.ds(base, B_PER_W)], idx_u)
    pltpu.sync_copy(nnid.at[pl.ds(base, B_PER_W)], idx_v)
    cp_u = pltpu.async_copy(table.at[idx_u], rows_u, sem_u)
    cp_v = pltpu.async_copy(table.at[idx_v], rows_v, sem_v)
    cp_b = pltpu.async_copy(bias_tab.at[idx_v], bias_vm, sem_b)
    cp_u.wait()
    cp_v.wait()
    cp_b.wait()

    lane = lax.iota(jnp.int32, NUM_LANES)

    def block(blk, carry):
        for i in range(NUM_LANES):
            row = blk * NUM_LANES + i
            p = rows_u[row, :] * rows_v[row, :]
            prod_vm[pl.ds(row * EMBED_DIM, EMBED_DIM)] = p
        flat_base = (lane + blk * NUM_LANES) * EMBED_DIM
        acc = jnp.zeros((NUM_LANES,), jnp.float32)
        for d in range(EMBED_DIM):
            acc = acc + plsc.load_gather(prod_vm, [flat_base + d])
        b = bias_vm[pl.ds(blk * NUM_LANES, NUM_LANES)]
        s = jnp.clip(acc + b, -10.0, 10.0)
        score_vm[pl.ds(blk * NUM_LANES, NUM_LANES)] = s
        return carry

    lax.fori_loop(0, NBLK, block, 0)

    pltpu.sync_copy(score_vm, score_out.at[pl.ds(base, B_PER_W)])
    pltpu.sync_copy(rows_u, embu_out.at[pl.ds(base, B_PER_W)])
    pltpu.sync_copy(rows_v, embv_out.at[pl.ds(base, B_PER_W)])
    pltpu.sync_copy(bias_vm, bias_out.at[pl.ds(base, B_PER_W)])


@jax.jit
def kernel(embedding_matrix, bias_vector, node_id, node_neighbor_id):
    mesh = plsc.VectorSubcoreMesh(core_axis_name="c", subcore_axis_name="s")
    f = functools.partial(
        pl.kernel,
        mesh=mesh,
        compiler_params=pltpu.CompilerParams(
            needs_layout_passes=False, use_tc_tiling_on_sc=False),
        out_type=[
            jax.ShapeDtypeStruct((BATCH,), jnp.float32),
            jax.ShapeDtypeStruct((BATCH, EMBED_DIM), jnp.float32),
            jax.ShapeDtypeStruct((BATCH, EMBED_DIM), jnp.float32),
            jax.ShapeDtypeStruct((BATCH,), jnp.float32),
        ],
        scratch_types=[
            pltpu.VMEM((B_PER_W,), jnp.int32),
            pltpu.VMEM((B_PER_W,), jnp.int32),
            pltpu.VMEM((B_PER_W, EMBED_DIM), jnp.float32),
            pltpu.VMEM((B_PER_W, EMBED_DIM), jnp.float32),
            pltpu.VMEM((B_PER_W * EMBED_DIM,), jnp.float32),
            pltpu.VMEM((B_PER_W,), jnp.float32),
            pltpu.VMEM((B_PER_W,), jnp.float32),
            pltpu.SemaphoreType.DMA,
            pltpu.SemaphoreType.DMA,
            pltpu.SemaphoreType.DMA,
        ],
    )(_sc_body)
    score, embu, embv, bias = f(
        embedding_matrix,
        bias_vector,
        node_id.astype(jnp.int32),
        node_neighbor_id.astype(jnp.int32),
    )
    return (score, embu, embv, bias)
